# Initial kernel scaffold; baseline (speedup 1.0000x reference)
import jax
import jax.numpy as jnp
from jax.experimental import pallas as pl


def kernel(x, edge_index, batch, W1, b1, Wp1, bp1, W2, b2, Wp2, bp2, lin1_W, lin1_b, lin2_W, lin2_b):
    raise NotImplementedError("write your pallas kernel here")


# speedup vs baseline: 13.1441x; 13.1441x over previous
import functools

import jax
import jax.numpy as jnp
from jax import lax
from jax.experimental import pallas as pl
from jax.experimental.pallas import tpu as pltpu
from jax.experimental.pallas import tpu_sc as plsc

B = 64
NEG = -1e30
RB = 512
CH = 128
NW = 32


def _sc_edge_agg(table, src, dst, zeros, *, D, NPAD, stride):
    EPAD = src.shape[0]
    n_chunks = EPAD // CH
    t_max = (n_chunks + NW - 1) // NW
    RPT = NPAD // 16

    mesh = plsc.VectorSubcoreMesh(core_axis_name="c", subcore_axis_name="s")

    @functools.partial(
        pl.kernel,
        out_type=jax.ShapeDtypeStruct((2, NPAD, D), jnp.float32),
        mesh=mesh,
        scratch_types=[
            pltpu.VMEM((1, CH), jnp.int32),
            pltpu.VMEM((1, CH), jnp.int32),
            pltpu.VMEM((CH, D), jnp.float32),
            pltpu.VMEM_SHARED((NPAD, D), jnp.float32),
            pltpu.SemaphoreType.DMA,
        ],
    )
    def k(table_hbm, src_hbm, dst_hbm, zeros_hbm, out_hbm, idx_s, idx_d, rows, acc, sem):
        c = lax.axis_index("c")
        s = lax.axis_index("s")
        w = c * 16 + s
        pltpu.sync_copy(zeros_hbm, acc.at[pl.ds(s * RPT, RPT)])
        plsc.subcore_barrier()

        def body(t, carry):
            j = w + NW * t

            @pl.when(j < n_chunks)
            def _():
                base = j * CH
                pltpu.sync_copy(src_hbm.at[pl.ds(base, CH)], idx_s.at[0])
                pltpu.sync_copy(dst_hbm.at[pl.ds(base, CH)], idx_d.at[0])
                if stride != 1:
                    for q in range(CH // 16):
                        v = idx_s[0, pl.ds(q * 16, 16)]
                        idx_s[0, pl.ds(q * 16, 16)] = v * stride
                pltpu.async_copy(table_hbm.at[idx_s.at[0]], rows, sem).wait()
                pltpu.sync_copy(rows, acc.at[idx_d.at[0]], add=True)

            return carry

        lax.fori_loop(0, t_max, body, 0)
        plsc.subcore_barrier()
        pltpu.sync_copy(acc.at[pl.ds(s * RPT, RPT)], out_hbm.at[c, pl.ds(s * RPT, RPT)])

    return k(table, src, dst, zeros)


def _prep_body(degp_ref, base_ref, sm_ref, brow_ref, u_ref, dinv_ref, kmeta_ref):
    i = pl.program_id(0)
    p = degp_ref[0] + degp_ref[1]
    deg = sm_ref[...] * (p + 1.0)
    dinv = jnp.where(deg > 0, 1.0 / jnp.sqrt(jnp.maximum(deg, 1e-12)), 0.0)
    dinv_ref[...] = dinv
    u_ref[...] = dinv * base_ref[...]

    @pl.when(i == 0)
    def _():
        g = lax.broadcasted_iota(jnp.int32, (B, 1), 0)
        cnt = jnp.sum((brow_ref[...] == g).astype(jnp.float32), axis=1, keepdims=True)
        k1 = jnp.floor((cnt + 1.0) * 0.5)
        k2 = jnp.floor((k1 + 1.0) * 0.5)
        kmeta_ref[:, 0:1] = k1
        kmeta_ref[:, 1:2] = k2


def _tc_prep(degp, basef, selfm, brow, NPAD, H):
    nblk = NPAD // RB
    return pl.pallas_call(
        _prep_body,
        grid=(nblk,),
        in_specs=[
            pl.BlockSpec((2, RB, H), lambda i: (0, i, 0)),
            pl.BlockSpec((RB, H), lambda i: (i, 0)),
            pl.BlockSpec((RB, H), lambda i: (i, 0)),
            pl.BlockSpec((1, NPAD), lambda i: (0, 0)),
        ],
        out_specs=[
            pl.BlockSpec((RB, H), lambda i: (i, 0)),
            pl.BlockSpec((RB, H), lambda i: (i, 0)),
            pl.BlockSpec((B, 2), lambda i: (0, 0)),
        ],
        out_shape=[
            jax.ShapeDtypeStruct((NPAD, H), jnp.float32),
            jax.ShapeDtypeStruct((NPAD, H), jnp.float32),
            jax.ShapeDtypeStruct((B, 2), jnp.float32),
        ],
    )(degp, basef, selfm, brow)


def _conv_body(aggp_ref, base_ref, dinv_ref, w_ref, b_ref, wp_ref, h_ref, sw_ref, vw_ref):
    dinv = dinv_ref[...]
    agg = dinv * (aggp_ref[0] + aggp_ref[1]) + dinv * dinv * base_ref[...]
    h = jnp.maximum(jnp.dot(agg, w_ref[...], preferred_element_type=jnp.float32)
                    + b_ref[...], 0.0)
    h_ref[...] = h
    s = jnp.sum(h * wp_ref[...], axis=1, keepdims=True)
    sw = jnp.broadcast_to(s, h.shape)
    sw_ref[...] = sw
    vw_ref[...] = dinv * sw


def _tc_conv(aggp, basef, dinvw, W, b_row, wp_row, NPAD, H):
    nblk = NPAD // RB
    return pl.pallas_call(
        _conv_body,
        grid=(nblk,),
        in_specs=[
            pl.BlockSpec((2, RB, H), lambda i: (0, i, 0)),
            pl.BlockSpec((RB, H), lambda i: (i, 0)),
            pl.BlockSpec((RB, H), lambda i: (i, 0)),
            pl.BlockSpec((H, H), lambda i: (0, 0)),
            pl.BlockSpec((1, H), lambda i: (0, 0)),
            pl.BlockSpec((1, H), lambda i: (0, 0)),
        ],
        out_specs=[
            pl.BlockSpec((RB, H), lambda i: (i, 0)),
            pl.BlockSpec((RB, H), lambda i: (i, 0)),
            pl.BlockSpec((RB, H), lambda i: (i, 0)),
        ],
        out_shape=[
            jax.ShapeDtypeStruct((NPAD, H), jnp.float32),
            jax.ShapeDtypeStruct((NPAD, H), jnp.float32),
            jax.ShapeDtypeStruct((NPAD, H), jnp.float32),
        ],
    )(aggp, basef, dinvw, W, b_row, wp_row)


def _score_body(scp_ref, dinv_ref, sw_ref, bp_ref, mprev_ref, sm_ref, st_ref):
    dinv = dinv_ref[...]
    p = scp_ref[0] + scp_ref[1]
    score = dinv * p + dinv * dinv * sw_ref[...] + bp_ref[0, 0]
    mp = mprev_ref[...]
    sm_ref[...] = jnp.where(mp > 0, score, NEG)
    st_ref[...] = jnp.where(mp > 0, score, 0.0)


def _tc_score(scp, dinvw, sw, bp, mprev, NPAD, H):
    nblk = NPAD // RB
    return pl.pallas_call(
        _score_body,
        grid=(nblk,),
        in_specs=[
            pl.BlockSpec((2, RB, H), lambda i: (0, i, 0)),
            pl.BlockSpec((RB, H), lambda i: (i, 0)),
            pl.BlockSpec((RB, H), lambda i: (i, 0)),
            pl.BlockSpec((1, 1), lambda i: (0, 0)),
            pl.BlockSpec((RB, H), lambda i: (i, 0)),
        ],
        out_specs=[
            pl.BlockSpec((RB, H), lambda i: (i, 0)),
            pl.BlockSpec((RB, H), lambda i: (i, 0)),
        ],
        out_shape=[
            jax.ShapeDtypeStruct((NPAD, H), jnp.float32),
            jax.ShapeDtypeStruct((NPAD, H), jnp.float32),
        ],
    )(scp, dinvw, sw, bp, mprev)


def _rank_body(scol_ref, srow_ref, bcol_ref, brow_ref, kcol_ref, basem_ref, mask_ref):
    i = pl.program_id(0)
    s_i = scol_ref[:, 0:1]
    b_i = bcol_ref[...]
    idx_i = lax.broadcasted_iota(jnp.int32, (RB, 1), 0) + i * RB
    brow = brow_ref[...]
    bmin = jnp.min(b_i)
    bmax = jnp.max(b_i)
    jlo = jnp.sum((brow < bmin).astype(jnp.int32))
    jhi = jnp.sum((brow <= bmax).astype(jnp.int32))
    CJ = 1024
    clo = jlo // CJ
    chi = (jhi + CJ - 1) // CJ

    def jbody(cc, r):
        s_j = srow_ref[0:1, pl.ds(cc * CJ, CJ)]
        b_j = brow_ref[0:1, pl.ds(cc * CJ, CJ)]
        idx_j = lax.broadcasted_iota(jnp.int32, (1, CJ), 1) + cc * CJ
        same = b_j == b_i
        beat = (s_j > s_i) | ((s_j == s_i) & (idx_j < idx_i))
        return r + jnp.sum(jnp.where(same & beat, 1.0, 0.0), axis=1, keepdims=True)

    rank = lax.fori_loop(clo, chi, jbody, jnp.zeros((RB, 1), jnp.float32))
    g = lax.broadcasted_iota(jnp.int32, (1, B), 1)
    onehot = (b_i == g).astype(jnp.float32)
    kv = jnp.dot(onehot, kcol_ref[...], preferred_element_type=jnp.float32)
    ind = jnp.where(rank < kv, 1.0, 0.0)
    mask_ref[...] = jnp.broadcast_to(ind, mask_ref.shape) * basem_ref[...]


def _tc_rank(scol, srow, bcol, brow, kcol, basem, NPAD, H):
    nblk = NPAD // RB
    return pl.pallas_call(
        _rank_body,
        grid=(nblk,),
        in_specs=[
            pl.BlockSpec((RB, H), lambda i: (i, 0)),
            pl.BlockSpec((1, NPAD), lambda i: (0, 0)),
            pl.BlockSpec((RB, 1), lambda i: (i, 0)),
            pl.BlockSpec((1, NPAD), lambda i: (0, 0)),
            pl.BlockSpec((B, 1), lambda i: (0, 0)),
            pl.BlockSpec((RB, H), lambda i: (i, 0)),
        ],
        out_specs=pl.BlockSpec((RB, H), lambda i: (i, 0)),
        out_shape=jax.ShapeDtypeStruct((NPAD, H), jnp.float32),
    )(scol, srow, bcol, brow, kcol, basem)


def _pool_body(h_ref, st_ref, m_ref, bcol_ref, hp_ref, xcat_ref, smax, ssum, scnt):
    i = pl.program_id(0)
    nblk = pl.num_programs(0)

    @pl.when(i == 0)
    def _():
        smax[...] = jnp.full(smax.shape, NEG, jnp.float32)
        ssum[...] = jnp.zeros(ssum.shape, jnp.float32)
        scnt[...] = jnp.zeros(scnt.shape, jnp.float32)

    m = m_ref[...]
    hp = h_ref[...] * jnp.tanh(st_ref[...]) * m
    hp_ref[...] = hp
    b_i = bcol_ref[...]
    bmin = jnp.min(b_i)
    bhi = jnp.minimum(jnp.max(b_i), B - 1)

    def gbody(g, carry):
        gm = (b_i == g).astype(jnp.float32)
        sel = gm * m[:, 0:1]
        val = jnp.where(sel > 0, hp, NEG)
        mx = jnp.max(val, axis=0, keepdims=True)
        sm = jnp.sum(hp * gm, axis=0, keepdims=True)
        cn = jnp.sum(sel)
        smax[pl.ds(g, 1), :] = jnp.maximum(smax[pl.ds(g, 1), :], mx)
        ssum[pl.ds(g, 1), :] = ssum[pl.ds(g, 1), :] + sm
        scnt[pl.ds(g, 1), :] = scnt[pl.ds(g, 1), :] + cn
        return carry

    lax.fori_loop(bmin, bhi + 1, gbody, 0)

    @pl.when(i == nblk - 1)
    def _():
        gmax = jnp.where(smax[...] <= -1e29, 0.0, smax[...])
        gmean = ssum[...] / jnp.maximum(scnt[...], 1.0)
        hcols = xcat_ref.shape[1] // 2
        xcat_ref[:, 0:hcols] = gmax
        xcat_ref[:, hcols:] = gmean


def _tc_pool(h, st, m, bcol, NPAD, H):
    nblk = NPAD // RB
    return pl.pallas_call(
        _pool_body,
        grid=(nblk,),
        in_specs=[
            pl.BlockSpec((RB, H), lambda i: (i, 0)),
            pl.BlockSpec((RB, H), lambda i: (i, 0)),
            pl.BlockSpec((RB, H), lambda i: (i, 0)),
            pl.BlockSpec((RB, 1), lambda i: (i, 0)),
        ],
        out_specs=[
            pl.BlockSpec((RB, H), lambda i: (i, 0)),
            pl.BlockSpec((B, 2 * H), lambda i: (0, 0)),
        ],
        out_shape=[
            jax.ShapeDtypeStruct((NPAD, H), jnp.float32),
            jax.ShapeDtypeStruct((B, 2 * H), jnp.float32),
        ],
        scratch_shapes=[
            pltpu.VMEM((B, H), jnp.float32),
            pltpu.VMEM((B, H), jnp.float32),
            pltpu.VMEM((B, H), jnp.float32),
        ],
    )(h, st, m, bcol)


def _final_body(x1_ref, x2_ref, w1_ref, b1_ref, w2_ref, b2_ref, out_ref):
    z = x1_ref[...] + x2_ref[...]
    z1 = jnp.maximum(jnp.dot(z, w1_ref[...], preferred_element_type=jnp.float32)
                     + b1_ref[...], 0.0)
    o = jnp.sum(z1 * w2_ref[...], axis=1, keepdims=True) + b2_ref[0, 0]
    out_ref[...] = o


def _tc_final(x1, x2, l1w, l1b, l2row, l2b, H):
    return pl.pallas_call(
        _final_body,
        out_shape=jax.ShapeDtypeStruct((B, 1), jnp.float32),
    )(x1, x2, l1w, l1b, l2row, l2b)


def kernel(x, edge_index, batch, W1, b1, Wp1, bp1, W2, b2, Wp2, bp2,
           lin1_W, lin1_b, lin2_W, lin2_b):
    N, F = x.shape
    H = W1.shape[1]
    E = edge_index.shape[1]
    NPAD = ((N + RB - 1) // RB) * RB
    EPAD = ((E + CH - 1) // CH) * CH
    RPT = NPAD // 16

    f32 = jnp.float32
    i32 = jnp.int32

    src = edge_index[0].astype(i32)
    dst = edge_index[1].astype(i32)
    if EPAD != E:
        src = jnp.concatenate([src, jnp.zeros((EPAD - E,), i32)])
        dst = jnp.concatenate([dst, jnp.full((EPAD - E,), NPAD - 1, i32)])

    x_pad = jnp.pad(x.astype(f32), ((0, NPAD - N), (0, 0)))
    batch_pad = jnp.concatenate(
        [batch.astype(i32), jnp.full((NPAD - N,), B, i32)])
    bcol = batch_pad.reshape(NPAD, 1)
    brow = batch_pad.reshape(1, NPAD)

    ones_w = jnp.ones((NPAD, H), f32)
    zerosH = jnp.zeros((RPT, H), f32)

    b1r = b1.reshape(1, H)
    b2r = b2.reshape(1, H)
    wp1r = Wp1.reshape(1, H)
    wp2r = Wp2.reshape(1, H)
    bp1r = bp1.reshape(1, 1)
    bp2r = bp2.reshape(1, 1)
    l1br = lin1_b.reshape(1, H)
    l2r = lin2_W.reshape(1, H)
    l2br = lin2_b.reshape(1, 1)

    degp = _sc_edge_agg(ones_w, src, dst, zerosH, D=H, NPAD=NPAD, stride=1)
    u1, dinv1, kmeta = _tc_prep(degp, x_pad, ones_w, brow, NPAD, H)
    aggp1 = _sc_edge_agg(u1, src, dst, zerosH, D=H, NPAD=NPAD, stride=1)
    h1, s1w, v1w = _tc_conv(aggp1, x_pad, dinv1, W1, b1r, wp1r, NPAD, H)
    scp1 = _sc_edge_agg(v1w, src, dst, zerosH, D=H, NPAD=NPAD, stride=1)
    sm1, st1 = _tc_score(scp1, dinv1, s1w, bp1r, ones_w, NPAD, H)
    k1col = kmeta[:, 0:1]
    k2col = kmeta[:, 1:2]
    sm1row = sm1[:, 0].reshape(1, NPAD)
    mask1 = _tc_rank(sm1, sm1row, bcol, brow, k1col, ones_w, NPAD, H)
    hp, x1cat = _tc_pool(h1, st1, mask1, bcol, NPAD, H)

    maccp = _sc_edge_agg(mask1, src, dst, zerosH, D=H, NPAD=NPAD, stride=1)
    u2, dinv2, _ = _tc_prep(maccp, hp, mask1, brow, NPAD, H)
    aggp2 = _sc_edge_agg(u2, src, dst, zerosH, D=H, NPAD=NPAD, stride=1)
    h2, s2w, v2w = _tc_conv(aggp2, hp, dinv2, W2, b2r, wp2r, NPAD, H)
    scp2 = _sc_edge_agg(v2w, src, dst, zerosH, D=H, NPAD=NPAD, stride=1)
    sm2, st2 = _tc_score(scp2, dinv2, s2w, bp2r, mask1, NPAD, H)
    sm2row = sm2[:, 0].reshape(1, NPAD)
    mask2 = _tc_rank(sm2, sm2row, bcol, brow, k2col, mask1, NPAD, H)
    _, x2cat = _tc_pool(h2, st2, mask2, bcol, NPAD, H)

    out = _tc_final(x1cat, x2cat, lin1_W, l1br, l2r, l2br, H)
    return out[:, 0]


# speedup vs baseline: 19.5560x; 1.4878x over previous
import functools

import jax
import jax.numpy as jnp
from jax import lax
from jax.experimental import pallas as pl
from jax.experimental.pallas import tpu as pltpu
from jax.experimental.pallas import tpu_sc as plsc

B = 64
NEG = -1e30
RB = 512
CH = 128
NW = 32


def _sc_edge_agg(table, src, dst, zeros, *, D, NPAD, stride):
    EPAD = src.shape[0]
    n_chunks = EPAD // CH
    t_max = (n_chunks + NW - 1) // NW
    RPT = NPAD // 16

    mesh = plsc.VectorSubcoreMesh(core_axis_name="c", subcore_axis_name="s")

    @functools.partial(
        pl.kernel,
        out_type=jax.ShapeDtypeStruct((2, NPAD, D), jnp.float32),
        mesh=mesh,
        scratch_types=[
            pltpu.VMEM((1, CH), jnp.int32),
            pltpu.VMEM((1, CH), jnp.int32),
            pltpu.VMEM((CH, D), jnp.float32),
            pltpu.VMEM_SHARED((NPAD, D), jnp.float32),
            pltpu.SemaphoreType.DMA,
        ],
    )
    def k(table_hbm, src_hbm, dst_hbm, zeros_hbm, out_hbm, idx_s, idx_d, rows, acc, sem):
        c = lax.axis_index("c")
        s = lax.axis_index("s")
        w = c * 16 + s
        pltpu.sync_copy(zeros_hbm, acc.at[pl.ds(s * RPT, RPT)])
        plsc.subcore_barrier()

        def body(t, carry):
            j = w + NW * t

            @pl.when(j < n_chunks)
            def _():
                base = j * CH
                pltpu.sync_copy(src_hbm.at[pl.ds(base, CH)], idx_s.at[0])
                pltpu.sync_copy(dst_hbm.at[pl.ds(base, CH)], idx_d.at[0])
                if stride != 1:
                    for q in range(CH // 16):
                        v = idx_s[0, pl.ds(q * 16, 16)]
                        idx_s[0, pl.ds(q * 16, 16)] = v * stride
                pltpu.async_copy(table_hbm.at[idx_s.at[0]], rows, sem).wait()
                pltpu.sync_copy(rows, acc.at[idx_d.at[0]], add=True)

            return carry

        lax.fori_loop(0, t_max, body, 0)
        plsc.subcore_barrier()
        pltpu.sync_copy(acc.at[pl.ds(s * RPT, RPT)], out_hbm.at[c, pl.ds(s * RPT, RPT)])

    return k(table, src, dst, zeros)


def _sc_scalar_agg(table, src, dst, zeros_n, *, NPAD):
    EPAD = src.shape[0]
    n_chunks = EPAD // CH
    t_max = (n_chunks + NW - 1) // NW
    RPT = NPAD // 16

    mesh = plsc.VectorSubcoreMesh(core_axis_name="c", subcore_axis_name="s")

    @functools.partial(
        pl.kernel,
        out_type=jax.ShapeDtypeStruct((2, NPAD), jnp.float32),
        mesh=mesh,
        compiler_params=pltpu.CompilerParams(needs_layout_passes=False),
        scratch_types=[
            pltpu.VMEM((1, CH), jnp.int32),
            pltpu.VMEM((1, CH), jnp.int32),
            pltpu.VMEM((NPAD,), jnp.float32),
            pltpu.VMEM((NPAD,), jnp.float32),
            pltpu.VMEM((16, RPT), jnp.float32),
            pltpu.VMEM_SHARED((16, NPAD), jnp.float32),
        ],
    )
    def k(table_hbm, src_hbm, dst_hbm, zeros_hbm, out_hbm,
          idx_s, idx_d, tab, acc, red, shared):
        c = lax.axis_index("c")
        s = lax.axis_index("s")
        w = c * 16 + s
        pltpu.sync_copy(table_hbm, tab)
        pltpu.sync_copy(zeros_hbm, acc)

        def body(t, carry):
            j = w + NW * t

            @pl.when(j < n_chunks)
            def _():
                base = j * CH
                pltpu.sync_copy(src_hbm.at[pl.ds(base, CH)], idx_s.at[0])
                pltpu.sync_copy(dst_hbm.at[pl.ds(base, CH)], idx_d.at[0])
                for q in range(CH // 16):
                    sidx = idx_s[0, pl.ds(q * 16, 16)]
                    didx = idx_d[0, pl.ds(q * 16, 16)]
                    vals = plsc.load_gather(tab, [sidx])
                    plsc.addupdate_scatter(acc, [didx], vals)

            return carry

        lax.fori_loop(0, t_max, body, 0)
        pltpu.sync_copy(acc, shared.at[s])
        plsc.subcore_barrier()
        pltpu.sync_copy(shared.at[:, pl.ds(s * RPT, RPT)], red)
        for jj in range(RPT // 16):
            v = red[0, pl.ds(jj * 16, 16)]
            for r in range(1, 16):
                v = v + red[r, pl.ds(jj * 16, 16)]
            red[0, pl.ds(jj * 16, 16)] = v
        pltpu.sync_copy(red.at[0], out_hbm.at[c, pl.ds(s * RPT, RPT)])

    return k(table, src, dst, zeros_n)


def _prep_body(degp_ref, base_ref, sm_ref, brow_ref, u_ref, dinv_ref, kmeta_ref):
    i = pl.program_id(0)
    p = degp_ref[0] + degp_ref[1]
    deg = sm_ref[...] * (p + 1.0)
    dinv = jnp.where(deg > 0, 1.0 / jnp.sqrt(jnp.maximum(deg, 1e-12)), 0.0)
    dinv_ref[...] = dinv
    u_ref[...] = dinv * base_ref[...]

    @pl.when(i == 0)
    def _():
        g = lax.broadcasted_iota(jnp.int32, (B, 1), 0)
        cnt = jnp.sum((brow_ref[...] == g).astype(jnp.float32), axis=1, keepdims=True)
        k1 = jnp.floor((cnt + 1.0) * 0.5)
        k2 = jnp.floor((k1 + 1.0) * 0.5)
        kmeta_ref[:, 0:1] = k1
        kmeta_ref[:, 1:2] = k2


def _tc_prep(degp, basef, selfm, brow, NPAD, H):
    nblk = NPAD // RB
    return pl.pallas_call(
        _prep_body,
        grid=(nblk,),
        in_specs=[
            pl.BlockSpec((2, RB, 1), lambda i: (0, i, 0)),
            pl.BlockSpec((RB, H), lambda i: (i, 0)),
            pl.BlockSpec((RB, H), lambda i: (i, 0)),
            pl.BlockSpec((1, NPAD), lambda i: (0, 0)),
        ],
        out_specs=[
            pl.BlockSpec((RB, H), lambda i: (i, 0)),
            pl.BlockSpec((RB, H), lambda i: (i, 0)),
            pl.BlockSpec((B, 2), lambda i: (0, 0)),
        ],
        out_shape=[
            jax.ShapeDtypeStruct((NPAD, H), jnp.float32),
            jax.ShapeDtypeStruct((NPAD, H), jnp.float32),
            jax.ShapeDtypeStruct((B, 2), jnp.float32),
        ],
    )(degp, basef, selfm, brow)


def _conv_body(aggp_ref, base_ref, dinv_ref, w_ref, b_ref, wp_ref, h_ref, sw_ref, vw_ref):
    dinv = dinv_ref[...]
    agg = dinv * (aggp_ref[0] + aggp_ref[1]) + dinv * dinv * base_ref[...]
    h = jnp.maximum(jnp.dot(agg, w_ref[...], preferred_element_type=jnp.float32)
                    + b_ref[...], 0.0)
    h_ref[...] = h
    s = jnp.sum(h * wp_ref[...], axis=1, keepdims=True)
    sw = jnp.broadcast_to(s, h.shape)
    sw_ref[...] = sw
    vw_ref[...] = dinv * sw


def _tc_conv(aggp, basef, dinvw, W, b_row, wp_row, NPAD, H):
    nblk = NPAD // RB
    return pl.pallas_call(
        _conv_body,
        grid=(nblk,),
        in_specs=[
            pl.BlockSpec((2, RB, H), lambda i: (0, i, 0)),
            pl.BlockSpec((RB, H), lambda i: (i, 0)),
            pl.BlockSpec((RB, H), lambda i: (i, 0)),
            pl.BlockSpec((H, H), lambda i: (0, 0)),
            pl.BlockSpec((1, H), lambda i: (0, 0)),
            pl.BlockSpec((1, H), lambda i: (0, 0)),
        ],
        out_specs=[
            pl.BlockSpec((RB, H), lambda i: (i, 0)),
            pl.BlockSpec((RB, H), lambda i: (i, 0)),
            pl.BlockSpec((RB, H), lambda i: (i, 0)),
        ],
        out_shape=[
            jax.ShapeDtypeStruct((NPAD, H), jnp.float32),
            jax.ShapeDtypeStruct((NPAD, H), jnp.float32),
            jax.ShapeDtypeStruct((NPAD, H), jnp.float32),
        ],
    )(aggp, basef, dinvw, W, b_row, wp_row)


def _score_body(scp_ref, dinv_ref, sw_ref, bp_ref, mprev_ref, sm_ref, st_ref):
    dinv = dinv_ref[...]
    p = scp_ref[0] + scp_ref[1]
    score = dinv * p + dinv * dinv * sw_ref[...] + bp_ref[0, 0]
    mp = mprev_ref[...]
    sm_ref[...] = jnp.where(mp > 0, score, NEG)
    st_ref[...] = jnp.where(mp > 0, score, 0.0)


def _tc_score(scp, dinvw, sw, bp, mprev, NPAD, H):
    nblk = NPAD // RB
    return pl.pallas_call(
        _score_body,
        grid=(nblk,),
        in_specs=[
            pl.BlockSpec((2, RB, 1), lambda i: (0, i, 0)),
            pl.BlockSpec((RB, H), lambda i: (i, 0)),
            pl.BlockSpec((RB, H), lambda i: (i, 0)),
            pl.BlockSpec((1, 1), lambda i: (0, 0)),
            pl.BlockSpec((RB, H), lambda i: (i, 0)),
        ],
        out_specs=[
            pl.BlockSpec((RB, H), lambda i: (i, 0)),
            pl.BlockSpec((RB, H), lambda i: (i, 0)),
        ],
        out_shape=[
            jax.ShapeDtypeStruct((NPAD, H), jnp.float32),
            jax.ShapeDtypeStruct((NPAD, H), jnp.float32),
        ],
    )(scp, dinvw, sw, bp, mprev)


def _rank_body(scol_ref, srow_ref, bcol_ref, brow_ref, kcol_ref, basem_ref, mask_ref):
    i = pl.program_id(0)
    s_i = scol_ref[:, 0:1]
    b_i = bcol_ref[...]
    idx_i = lax.broadcasted_iota(jnp.int32, (RB, 1), 0) + i * RB
    brow = brow_ref[...]
    bmin = jnp.min(b_i)
    bmax = jnp.max(b_i)
    jlo = jnp.sum((brow < bmin).astype(jnp.int32))
    jhi = jnp.sum((brow <= bmax).astype(jnp.int32))
    CJ = 1024
    clo = jlo // CJ
    chi = (jhi + CJ - 1) // CJ

    def jbody(cc, r):
        s_j = srow_ref[0:1, pl.ds(cc * CJ, CJ)]
        b_j = brow_ref[0:1, pl.ds(cc * CJ, CJ)]
        idx_j = lax.broadcasted_iota(jnp.int32, (1, CJ), 1) + cc * CJ
        same = b_j == b_i
        beat = (s_j > s_i) | ((s_j == s_i) & (idx_j < idx_i))
        return r + jnp.sum(jnp.where(same & beat, 1.0, 0.0), axis=1, keepdims=True)

    rank = lax.fori_loop(clo, chi, jbody, jnp.zeros((RB, 1), jnp.float32))
    g = lax.broadcasted_iota(jnp.int32, (1, B), 1)
    onehot = (b_i == g).astype(jnp.float32)
    kv = jnp.dot(onehot, kcol_ref[...], preferred_element_type=jnp.float32)
    ind = jnp.where(rank < kv, 1.0, 0.0)
    mask_ref[...] = jnp.broadcast_to(ind, mask_ref.shape) * basem_ref[...]


def _tc_rank(scol, srow, bcol, brow, kcol, basem, NPAD, H):
    nblk = NPAD // RB
    return pl.pallas_call(
        _rank_body,
        grid=(nblk,),
        in_specs=[
            pl.BlockSpec((RB, H), lambda i: (i, 0)),
            pl.BlockSpec((1, NPAD), lambda i: (0, 0)),
            pl.BlockSpec((RB, 1), lambda i: (i, 0)),
            pl.BlockSpec((1, NPAD), lambda i: (0, 0)),
            pl.BlockSpec((B, 1), lambda i: (0, 0)),
            pl.BlockSpec((RB, H), lambda i: (i, 0)),
        ],
        out_specs=pl.BlockSpec((RB, H), lambda i: (i, 0)),
        out_shape=jax.ShapeDtypeStruct((NPAD, H), jnp.float32),
    )(scol, srow, bcol, brow, kcol, basem)


def _pool_body(h_ref, st_ref, m_ref, bcol_ref, hp_ref, xcat_ref, smax, ssum, scnt):
    i = pl.program_id(0)
    nblk = pl.num_programs(0)

    @pl.when(i == 0)
    def _():
        smax[...] = jnp.full(smax.shape, NEG, jnp.float32)
        ssum[...] = jnp.zeros(ssum.shape, jnp.float32)
        scnt[...] = jnp.zeros(scnt.shape, jnp.float32)

    m = m_ref[...]
    hp = h_ref[...] * jnp.tanh(st_ref[...]) * m
    hp_ref[...] = hp
    b_i = bcol_ref[...]
    bmin = jnp.min(b_i)
    bhi = jnp.minimum(jnp.max(b_i), B - 1)

    def gbody(g, carry):
        gm = (b_i == g).astype(jnp.float32)
        sel = gm * m[:, 0:1]
        val = jnp.where(sel > 0, hp, NEG)
        mx = jnp.max(val, axis=0, keepdims=True)
        sm = jnp.sum(hp * gm, axis=0, keepdims=True)
        cn = jnp.sum(sel)
        smax[pl.ds(g, 1), :] = jnp.maximum(smax[pl.ds(g, 1), :], mx)
        ssum[pl.ds(g, 1), :] = ssum[pl.ds(g, 1), :] + sm
        scnt[pl.ds(g, 1), :] = scnt[pl.ds(g, 1), :] + cn
        return carry

    lax.fori_loop(bmin, bhi + 1, gbody, 0)

    @pl.when(i == nblk - 1)
    def _():
        gmax = jnp.where(smax[...] <= -1e29, 0.0, smax[...])
        gmean = ssum[...] / jnp.maximum(scnt[...], 1.0)
        hcols = xcat_ref.shape[1] // 2
        xcat_ref[:, 0:hcols] = gmax
        xcat_ref[:, hcols:] = gmean


def _tc_pool(h, st, m, bcol, NPAD, H):
    nblk = NPAD // RB
    return pl.pallas_call(
        _pool_body,
        grid=(nblk,),
        in_specs=[
            pl.BlockSpec((RB, H), lambda i: (i, 0)),
            pl.BlockSpec((RB, H), lambda i: (i, 0)),
            pl.BlockSpec((RB, H), lambda i: (i, 0)),
            pl.BlockSpec((RB, 1), lambda i: (i, 0)),
        ],
        out_specs=[
            pl.BlockSpec((RB, H), lambda i: (i, 0)),
            pl.BlockSpec((B, 2 * H), lambda i: (0, 0)),
        ],
        out_shape=[
            jax.ShapeDtypeStruct((NPAD, H), jnp.float32),
            jax.ShapeDtypeStruct((B, 2 * H), jnp.float32),
        ],
        scratch_shapes=[
            pltpu.VMEM((B, H), jnp.float32),
            pltpu.VMEM((B, H), jnp.float32),
            pltpu.VMEM((B, H), jnp.float32),
        ],
    )(h, st, m, bcol)


def _final_body(x1_ref, x2_ref, w1_ref, b1_ref, w2_ref, b2_ref, out_ref):
    z = x1_ref[...] + x2_ref[...]
    z1 = jnp.maximum(jnp.dot(z, w1_ref[...], preferred_element_type=jnp.float32)
                     + b1_ref[...], 0.0)
    o = jnp.sum(z1 * w2_ref[...], axis=1, keepdims=True) + b2_ref[0, 0]
    out_ref[...] = o


def _tc_final(x1, x2, l1w, l1b, l2row, l2b, H):
    return pl.pallas_call(
        _final_body,
        out_shape=jax.ShapeDtypeStruct((B, 1), jnp.float32),
    )(x1, x2, l1w, l1b, l2row, l2b)


def kernel(x, edge_index, batch, W1, b1, Wp1, bp1, W2, b2, Wp2, bp2,
           lin1_W, lin1_b, lin2_W, lin2_b):
    N, F = x.shape
    H = W1.shape[1]
    E = edge_index.shape[1]
    NPAD = ((N + RB - 1) // RB) * RB
    EPAD = ((E + CH - 1) // CH) * CH
    RPT = NPAD // 16

    f32 = jnp.float32
    i32 = jnp.int32

    src = edge_index[0].astype(i32)
    dst = edge_index[1].astype(i32)
    if EPAD != E:
        src = jnp.concatenate([src, jnp.zeros((EPAD - E,), i32)])
        dst = jnp.concatenate([dst, jnp.full((EPAD - E,), NPAD - 1, i32)])

    x_pad = jnp.pad(x.astype(f32), ((0, NPAD - N), (0, 0)))
    batch_pad = jnp.concatenate(
        [batch.astype(i32), jnp.full((NPAD - N,), B, i32)])
    bcol = batch_pad.reshape(NPAD, 1)
    brow = batch_pad.reshape(1, NPAD)

    ones_w = jnp.ones((NPAD, H), f32)
    ones_n = jnp.ones((NPAD,), f32)
    zeros_n = jnp.zeros((NPAD,), f32)
    zerosH = jnp.zeros((RPT, H), f32)

    b1r = b1.reshape(1, H)
    b2r = b2.reshape(1, H)
    wp1r = Wp1.reshape(1, H)
    wp2r = Wp2.reshape(1, H)
    bp1r = bp1.reshape(1, 1)
    bp2r = bp2.reshape(1, 1)
    l1br = lin1_b.reshape(1, H)
    l2r = lin2_W.reshape(1, H)
    l2br = lin2_b.reshape(1, 1)

    degp = _sc_scalar_agg(ones_n, src, dst, zeros_n, NPAD=NPAD).reshape(2, NPAD, 1)
    u1, dinv1, kmeta = _tc_prep(degp, x_pad, ones_w, brow, NPAD, H)
    aggp1 = _sc_edge_agg(u1, src, dst, zerosH, D=H, NPAD=NPAD, stride=1)
    h1, s1w, v1w = _tc_conv(aggp1, x_pad, dinv1, W1, b1r, wp1r, NPAD, H)
    scp1 = _sc_scalar_agg(v1w[:, 0], src, dst, zeros_n,
                          NPAD=NPAD).reshape(2, NPAD, 1)
    sm1, st1 = _tc_score(scp1, dinv1, s1w, bp1r, ones_w, NPAD, H)
    k1col = kmeta[:, 0:1]
    k2col = kmeta[:, 1:2]
    sm1row = sm1[:, 0].reshape(1, NPAD)
    mask1 = _tc_rank(sm1, sm1row, bcol, brow, k1col, ones_w, NPAD, H)
    hp, x1cat = _tc_pool(h1, st1, mask1, bcol, NPAD, H)

    maccp = _sc_scalar_agg(mask1[:, 0], src, dst, zeros_n,
                           NPAD=NPAD).reshape(2, NPAD, 1)
    u2, dinv2, _ = _tc_prep(maccp, hp, mask1, brow, NPAD, H)
    aggp2 = _sc_edge_agg(u2, src, dst, zerosH, D=H, NPAD=NPAD, stride=1)
    h2, s2w, v2w = _tc_conv(aggp2, hp, dinv2, W2, b2r, wp2r, NPAD, H)
    scp2 = _sc_scalar_agg(v2w[:, 0], src, dst, zeros_n,
                          NPAD=NPAD).reshape(2, NPAD, 1)
    sm2, st2 = _tc_score(scp2, dinv2, s2w, bp2r, mask1, NPAD, H)
    sm2row = sm2[:, 0].reshape(1, NPAD)
    mask2 = _tc_rank(sm2, sm2row, bcol, brow, k2col, mask1, NPAD, H)
    _, x2cat = _tc_pool(h2, st2, mask2, bcol, NPAD, H)

    out = _tc_final(x1cat, x2cat, lin1_W, l1br, l2r, l2br, H)
    return out[:, 0]


# speedup vs baseline: 34.0703x; 1.7422x over previous
import functools

import jax
import jax.numpy as jnp
from jax import lax
from jax.experimental import pallas as pl
from jax.experimental.pallas import tpu as pltpu
from jax.experimental.pallas import tpu_sc as plsc

B = 64
NEG = -1e30
RB = 512
CH = 128
NW = 32


def _sc_edge_agg(table, src2, dst2, zeros, *, D, NPAD, NCHK):
    RPT = NPAD // 16
    NSEG = (NCHK + 15) // 16
    segs = (NSEG + NW - 1) // NW

    mesh = plsc.VectorSubcoreMesh(core_axis_name="c", subcore_axis_name="s")

    @functools.partial(
        pl.kernel,
        out_type=jax.ShapeDtypeStruct((2, NPAD, D), jnp.float32),
        mesh=mesh,
        scratch_types=[
            pltpu.VMEM((16, CH), jnp.int32),
            pltpu.VMEM((16, CH), jnp.int32),
            pltpu.VMEM((CH, D), jnp.float32),
            pltpu.VMEM((CH, D), jnp.float32),
            pltpu.VMEM_SHARED((NPAD, D), jnp.float32),
            pltpu.SemaphoreType.DMA,
            pltpu.SemaphoreType.DMA,
        ],
    )
    def k(table_hbm, src_hbm, dst_hbm, zeros_hbm, out_hbm,
          idx_s, idx_d, rows0, rows1, acc, sem0, sem1):
        c = lax.axis_index("c")
        s = lax.axis_index("s")
        w = c * 16 + s
        slo = (w * NSEG) // NW
        shi = ((w + 1) * NSEG) // NW
        rows = [rows0, rows1]
        sems = [sem0, sem1]
        pltpu.sync_copy(zeros_hbm, acc.at[pl.ds(s * RPT, RPT)])
        plsc.subcore_barrier()

        def seg_body(sg, carry):
            si = slo + sg

            @pl.when(si < shi)
            def _():
                base = si * 16
                pltpu.sync_copy(src_hbm.at[pl.ds(base, 16)], idx_s)
                pltpu.sync_copy(dst_hbm.at[pl.ds(base, 16)], idx_d)
                descs = {}
                descs[0] = pltpu.async_copy(
                    table_hbm.at[idx_s.at[0]], rows[0], sems[0])
                for kk in range(16):
                    b = kk & 1

                    @pl.when(base + kk < NCHK)
                    def _(kk=kk, b=b):
                        if kk + 1 < 16:
                            @pl.when(base + kk + 1 < NCHK)
                            def _():
                                nb = (kk + 1) & 1
                                descs[nb] = pltpu.async_copy(
                                    table_hbm.at[idx_s.at[kk + 1]],
                                    rows[nb], sems[nb])
                        descs[b].wait()
                        pltpu.sync_copy(rows[b], acc.at[idx_d.at[kk]], add=True)

            return carry

        lax.fori_loop(0, segs, seg_body, 0)
        plsc.subcore_barrier()
        pltpu.sync_copy(acc.at[pl.ds(s * RPT, RPT)], out_hbm.at[c, pl.ds(s * RPT, RPT)])

    return k(table, src2, dst2, zeros)


def _sc_scalar_agg(table, src2, dst2, zeros_n, *, NPAD, NCHK):
    RPT = NPAD // 16
    NSEG = (NCHK + 15) // 16
    segs = (NSEG + NW - 1) // NW

    mesh = plsc.VectorSubcoreMesh(core_axis_name="c", subcore_axis_name="s")

    @functools.partial(
        pl.kernel,
        out_type=jax.ShapeDtypeStruct((2, NPAD), jnp.float32),
        mesh=mesh,
        compiler_params=pltpu.CompilerParams(needs_layout_passes=False),
        scratch_types=[
            pltpu.VMEM((16, CH), jnp.int32),
            pltpu.VMEM((16, CH), jnp.int32),
            pltpu.VMEM((NPAD,), jnp.float32),
            pltpu.VMEM((NPAD,), jnp.float32),
            pltpu.VMEM((16, RPT), jnp.float32),
            pltpu.VMEM_SHARED((16, NPAD), jnp.float32),
        ],
    )
    def k(table_hbm, src_hbm, dst_hbm, zeros_hbm, out_hbm,
          idx_s, idx_d, tab, acc, red, shared):
        c = lax.axis_index("c")
        s = lax.axis_index("s")
        w = c * 16 + s
        slo = (w * NSEG) // NW
        shi = ((w + 1) * NSEG) // NW
        pltpu.sync_copy(table_hbm, tab)
        pltpu.sync_copy(zeros_hbm, acc)

        def seg_body(sg, carry):
            si = slo + sg

            @pl.when(si < shi)
            def _():
                base = si * 16
                pltpu.sync_copy(src_hbm.at[pl.ds(base, 16)], idx_s)
                pltpu.sync_copy(dst_hbm.at[pl.ds(base, 16)], idx_d)
                for kk in range(16):
                    @pl.when(base + kk < NCHK)
                    def _(kk=kk):
                        for q in range(CH // 16):
                            sidx = idx_s[kk, pl.ds(q * 16, 16)]
                            didx = idx_d[kk, pl.ds(q * 16, 16)]
                            vals = plsc.load_gather(tab, [sidx])
                            plsc.addupdate_scatter(acc, [didx], vals)

            return carry

        lax.fori_loop(0, segs, seg_body, 0)
        pltpu.sync_copy(acc, shared.at[s])
        plsc.subcore_barrier()
        pltpu.sync_copy(shared.at[:, pl.ds(s * RPT, RPT)], red)
        for jj in range(RPT // 16):
            v = red[0, pl.ds(jj * 16, 16)]
            for r in range(1, 16):
                v = v + red[r, pl.ds(jj * 16, 16)]
            red[0, pl.ds(jj * 16, 16)] = v
        pltpu.sync_copy(red.at[0], out_hbm.at[c, pl.ds(s * RPT, RPT)])

    return k(table, src2, dst2, zeros_n)


def _prep_body(degp_ref, base_ref, sm_ref, brow_ref, u_ref, dinv_ref, kmeta_ref):
    i = pl.program_id(0)
    p = degp_ref[0] + degp_ref[1]
    deg = sm_ref[...] * (p + 1.0)
    dinv = jnp.where(deg > 0, 1.0 / jnp.sqrt(jnp.maximum(deg, 1e-12)), 0.0)
    dinv_ref[...] = dinv
    u_ref[...] = dinv * base_ref[...]

    @pl.when(i == 0)
    def _():
        g = lax.broadcasted_iota(jnp.int32, (B, 1), 0)
        cnt = jnp.sum((brow_ref[...] == g).astype(jnp.float32), axis=1, keepdims=True)
        k1 = jnp.floor((cnt + 1.0) * 0.5)
        k2 = jnp.floor((k1 + 1.0) * 0.5)
        kmeta_ref[:, 0:1] = k1
        kmeta_ref[:, 1:2] = k2


def _tc_prep(degp, basef, selfm, brow, NPAD, H):
    nblk = NPAD // RB
    return pl.pallas_call(
        _prep_body,
        grid=(nblk,),
        in_specs=[
            pl.BlockSpec((2, RB, 1), lambda i: (0, i, 0)),
            pl.BlockSpec((RB, H), lambda i: (i, 0)),
            pl.BlockSpec((RB, H), lambda i: (i, 0)),
            pl.BlockSpec((1, NPAD), lambda i: (0, 0)),
        ],
        out_specs=[
            pl.BlockSpec((RB, H), lambda i: (i, 0)),
            pl.BlockSpec((RB, H), lambda i: (i, 0)),
            pl.BlockSpec((B, 2), lambda i: (0, 0)),
        ],
        out_shape=[
            jax.ShapeDtypeStruct((NPAD, H), jnp.float32),
            jax.ShapeDtypeStruct((NPAD, H), jnp.float32),
            jax.ShapeDtypeStruct((B, 2), jnp.float32),
        ],
    )(degp, basef, selfm, brow)


def _conv_body(aggp_ref, base_ref, dinv_ref, w_ref, b_ref, wp_ref, h_ref, sw_ref, vw_ref):
    dinv = dinv_ref[...]
    agg = dinv * (aggp_ref[0] + aggp_ref[1]) + dinv * dinv * base_ref[...]
    h = jnp.maximum(jnp.dot(agg, w_ref[...], preferred_element_type=jnp.float32)
                    + b_ref[...], 0.0)
    h_ref[...] = h
    s = jnp.sum(h * wp_ref[...], axis=1, keepdims=True)
    sw = jnp.broadcast_to(s, h.shape)
    sw_ref[...] = sw
    vw_ref[...] = dinv * sw


def _tc_conv(aggp, basef, dinvw, W, b_row, wp_row, NPAD, H):
    nblk = NPAD // RB
    return pl.pallas_call(
        _conv_body,
        grid=(nblk,),
        in_specs=[
            pl.BlockSpec((2, RB, H), lambda i: (0, i, 0)),
            pl.BlockSpec((RB, H), lambda i: (i, 0)),
            pl.BlockSpec((RB, H), lambda i: (i, 0)),
            pl.BlockSpec((H, H), lambda i: (0, 0)),
            pl.BlockSpec((1, H), lambda i: (0, 0)),
            pl.BlockSpec((1, H), lambda i: (0, 0)),
        ],
        out_specs=[
            pl.BlockSpec((RB, H), lambda i: (i, 0)),
            pl.BlockSpec((RB, H), lambda i: (i, 0)),
            pl.BlockSpec((RB, H), lambda i: (i, 0)),
        ],
        out_shape=[
            jax.ShapeDtypeStruct((NPAD, H), jnp.float32),
            jax.ShapeDtypeStruct((NPAD, H), jnp.float32),
            jax.ShapeDtypeStruct((NPAD, H), jnp.float32),
        ],
    )(aggp, basef, dinvw, W, b_row, wp_row)


def _score_body(scp_ref, dinv_ref, sw_ref, bp_ref, mprev_ref, sm_ref, st_ref):
    dinv = dinv_ref[...]
    p = scp_ref[0] + scp_ref[1]
    score = dinv * p + dinv * dinv * sw_ref[...] + bp_ref[0, 0]
    mp = mprev_ref[...]
    sm_ref[...] = jnp.where(mp > 0, score, NEG)
    st_ref[...] = jnp.where(mp > 0, score, 0.0)


def _tc_score(scp, dinvw, sw, bp, mprev, NPAD, H):
    nblk = NPAD // RB
    return pl.pallas_call(
        _score_body,
        grid=(nblk,),
        in_specs=[
            pl.BlockSpec((2, RB, 1), lambda i: (0, i, 0)),
            pl.BlockSpec((RB, H), lambda i: (i, 0)),
            pl.BlockSpec((RB, H), lambda i: (i, 0)),
            pl.BlockSpec((1, 1), lambda i: (0, 0)),
            pl.BlockSpec((RB, H), lambda i: (i, 0)),
        ],
        out_specs=[
            pl.BlockSpec((RB, H), lambda i: (i, 0)),
            pl.BlockSpec((RB, H), lambda i: (i, 0)),
        ],
        out_shape=[
            jax.ShapeDtypeStruct((NPAD, H), jnp.float32),
            jax.ShapeDtypeStruct((NPAD, H), jnp.float32),
        ],
    )(scp, dinvw, sw, bp, mprev)


def _rank_body(scol_ref, srow_ref, bcol_ref, brow_ref, kcol_ref, basem_ref, mask_ref):
    i = pl.program_id(0)
    s_i = scol_ref[:, 0:1]
    b_i = bcol_ref[...]
    idx_i = lax.broadcasted_iota(jnp.int32, (RB, 1), 0) + i * RB
    brow = brow_ref[...]
    bmin = jnp.min(b_i)
    bmax = jnp.max(b_i)
    jlo = jnp.sum((brow < bmin).astype(jnp.int32))
    jhi = jnp.sum((brow <= bmax).astype(jnp.int32))
    CJ = 1024
    clo = jlo // CJ
    chi = (jhi + CJ - 1) // CJ

    def jbody(cc, r):
        s_j = srow_ref[0:1, pl.ds(cc * CJ, CJ)]
        b_j = brow_ref[0:1, pl.ds(cc * CJ, CJ)]
        idx_j = lax.broadcasted_iota(jnp.int32, (1, CJ), 1) + cc * CJ
        same = b_j == b_i
        beat = (s_j > s_i) | ((s_j == s_i) & (idx_j < idx_i))
        return r + jnp.sum(jnp.where(same & beat, 1.0, 0.0), axis=1, keepdims=True)

    rank = lax.fori_loop(clo, chi, jbody, jnp.zeros((RB, 1), jnp.float32))
    g = lax.broadcasted_iota(jnp.int32, (1, B), 1)
    onehot = (b_i == g).astype(jnp.float32)
    kv = jnp.dot(onehot, kcol_ref[...], preferred_element_type=jnp.float32)
    ind = jnp.where(rank < kv, 1.0, 0.0)
    mask_ref[...] = jnp.broadcast_to(ind, mask_ref.shape) * basem_ref[...]


def _tc_rank(scol, srow, bcol, brow, kcol, basem, NPAD, H):
    nblk = NPAD // RB
    return pl.pallas_call(
        _rank_body,
        grid=(nblk,),
        in_specs=[
            pl.BlockSpec((RB, H), lambda i: (i, 0)),
            pl.BlockSpec((1, NPAD), lambda i: (0, 0)),
            pl.BlockSpec((RB, 1), lambda i: (i, 0)),
            pl.BlockSpec((1, NPAD), lambda i: (0, 0)),
            pl.BlockSpec((B, 1), lambda i: (0, 0)),
            pl.BlockSpec((RB, H), lambda i: (i, 0)),
        ],
        out_specs=pl.BlockSpec((RB, H), lambda i: (i, 0)),
        out_shape=jax.ShapeDtypeStruct((NPAD, H), jnp.float32),
    )(scol, srow, bcol, brow, kcol, basem)


def _pool_body(h_ref, st_ref, m_ref, bcol_ref, hp_ref, xcat_ref, smax, ssum, scnt):
    i = pl.program_id(0)
    nblk = pl.num_programs(0)

    @pl.when(i == 0)
    def _():
        smax[...] = jnp.full(smax.shape, NEG, jnp.float32)
        ssum[...] = jnp.zeros(ssum.shape, jnp.float32)
        scnt[...] = jnp.zeros(scnt.shape, jnp.float32)

    m = m_ref[...]
    hp = h_ref[...] * jnp.tanh(st_ref[...]) * m
    hp_ref[...] = hp
    b_i = bcol_ref[...]
    bmin = jnp.min(b_i)
    bhi = jnp.minimum(jnp.max(b_i), B - 1)

    def gbody(g, carry):
        gm = (b_i == g).astype(jnp.float32)
        sel = gm * m[:, 0:1]
        val = jnp.where(sel > 0, hp, NEG)
        mx = jnp.max(val, axis=0, keepdims=True)
        sm = jnp.sum(hp * gm, axis=0, keepdims=True)
        cn = jnp.sum(sel)
        smax[pl.ds(g, 1), :] = jnp.maximum(smax[pl.ds(g, 1), :], mx)
        ssum[pl.ds(g, 1), :] = ssum[pl.ds(g, 1), :] + sm
        scnt[pl.ds(g, 1), :] = scnt[pl.ds(g, 1), :] + cn
        return carry

    lax.fori_loop(bmin, bhi + 1, gbody, 0)

    @pl.when(i == nblk - 1)
    def _():
        gmax = jnp.where(smax[...] <= -1e29, 0.0, smax[...])
        gmean = ssum[...] / jnp.maximum(scnt[...], 1.0)
        hcols = xcat_ref.shape[1] // 2
        xcat_ref[:, 0:hcols] = gmax
        xcat_ref[:, hcols:] = gmean


def _tc_pool(h, st, m, bcol, NPAD, H):
    nblk = NPAD // RB
    return pl.pallas_call(
        _pool_body,
        grid=(nblk,),
        in_specs=[
            pl.BlockSpec((RB, H), lambda i: (i, 0)),
            pl.BlockSpec((RB, H), lambda i: (i, 0)),
            pl.BlockSpec((RB, H), lambda i: (i, 0)),
            pl.BlockSpec((RB, 1), lambda i: (i, 0)),
        ],
        out_specs=[
            pl.BlockSpec((RB, H), lambda i: (i, 0)),
            pl.BlockSpec((B, 2 * H), lambda i: (0, 0)),
        ],
        out_shape=[
            jax.ShapeDtypeStruct((NPAD, H), jnp.float32),
            jax.ShapeDtypeStruct((B, 2 * H), jnp.float32),
        ],
        scratch_shapes=[
            pltpu.VMEM((B, H), jnp.float32),
            pltpu.VMEM((B, H), jnp.float32),
            pltpu.VMEM((B, H), jnp.float32),
        ],
    )(h, st, m, bcol)


def _final_body(x1_ref, x2_ref, w1_ref, b1_ref, w2_ref, b2_ref, out_ref):
    z = x1_ref[...] + x2_ref[...]
    z1 = jnp.maximum(jnp.dot(z, w1_ref[...], preferred_element_type=jnp.float32)
                     + b1_ref[...], 0.0)
    o = jnp.sum(z1 * w2_ref[...], axis=1, keepdims=True) + b2_ref[0, 0]
    out_ref[...] = o


def _tc_final(x1, x2, l1w, l1b, l2row, l2b, H):
    return pl.pallas_call(
        _final_body,
        out_shape=jax.ShapeDtypeStruct((B, 1), jnp.float32),
    )(x1, x2, l1w, l1b, l2row, l2b)


def kernel(x, edge_index, batch, W1, b1, Wp1, bp1, W2, b2, Wp2, bp2,
           lin1_W, lin1_b, lin2_W, lin2_b):
    N, F = x.shape
    H = W1.shape[1]
    E = edge_index.shape[1]
    NPAD = ((N + RB - 1) // RB) * RB
    EPAD = ((E + CH - 1) // CH) * CH
    RPT = NPAD // 16

    f32 = jnp.float32
    i32 = jnp.int32

    NCHK = EPAD // CH
    src = edge_index[0].astype(i32)
    dst = edge_index[1].astype(i32)
    src = jnp.concatenate([src, jnp.zeros((EPAD + 16 * CH - E,), i32)])
    dst = jnp.concatenate(
        [dst, jnp.full((EPAD + 16 * CH - E,), NPAD - 1, i32)])
    src = src.reshape(NCHK + 16, CH)
    dst = dst.reshape(NCHK + 16, CH)

    x_pad = jnp.pad(x.astype(f32), ((0, NPAD - N), (0, 0)))
    batch_pad = jnp.concatenate(
        [batch.astype(i32), jnp.full((NPAD - N,), B, i32)])
    bcol = batch_pad.reshape(NPAD, 1)
    brow = batch_pad.reshape(1, NPAD)

    ones_w = jnp.ones((NPAD, H), f32)
    ones_n = jnp.ones((NPAD,), f32)
    zeros_n = jnp.zeros((NPAD,), f32)
    zerosH = jnp.zeros((RPT, H), f32)

    b1r = b1.reshape(1, H)
    b2r = b2.reshape(1, H)
    wp1r = Wp1.reshape(1, H)
    wp2r = Wp2.reshape(1, H)
    bp1r = bp1.reshape(1, 1)
    bp2r = bp2.reshape(1, 1)
    l1br = lin1_b.reshape(1, H)
    l2r = lin2_W.reshape(1, H)
    l2br = lin2_b.reshape(1, 1)

    degp = _sc_scalar_agg(ones_n, src, dst, zeros_n, NPAD=NPAD, NCHK=NCHK).reshape(2, NPAD, 1)
    u1, dinv1, kmeta = _tc_prep(degp, x_pad, ones_w, brow, NPAD, H)
    aggp1 = _sc_edge_agg(u1, src, dst, zerosH, D=H, NPAD=NPAD, NCHK=NCHK)
    h1, s1w, v1w = _tc_conv(aggp1, x_pad, dinv1, W1, b1r, wp1r, NPAD, H)
    scp1 = _sc_scalar_agg(v1w[:, 0], src, dst, zeros_n,
                          NPAD=NPAD, NCHK=NCHK).reshape(2, NPAD, 1)
    sm1, st1 = _tc_score(scp1, dinv1, s1w, bp1r, ones_w, NPAD, H)
    k1col = kmeta[:, 0:1]
    k2col = kmeta[:, 1:2]
    sm1row = sm1[:, 0].reshape(1, NPAD)
    mask1 = _tc_rank(sm1, sm1row, bcol, brow, k1col, ones_w, NPAD, H)
    hp, x1cat = _tc_pool(h1, st1, mask1, bcol, NPAD, H)

    maccp = _sc_scalar_agg(mask1[:, 0], src, dst, zeros_n,
                           NPAD=NPAD, NCHK=NCHK).reshape(2, NPAD, 1)
    u2, dinv2, _ = _tc_prep(maccp, hp, mask1, brow, NPAD, H)
    aggp2 = _sc_edge_agg(u2, src, dst, zerosH, D=H, NPAD=NPAD, NCHK=NCHK)
    h2, s2w, v2w = _tc_conv(aggp2, hp, dinv2, W2, b2r, wp2r, NPAD, H)
    scp2 = _sc_scalar_agg(v2w[:, 0], src, dst, zeros_n,
                          NPAD=NPAD, NCHK=NCHK).reshape(2, NPAD, 1)
    sm2, st2 = _tc_score(scp2, dinv2, s2w, bp2r, mask1, NPAD, H)
    sm2row = sm2[:, 0].reshape(1, NPAD)
    mask2 = _tc_rank(sm2, sm2row, bcol, brow, k2col, mask1, NPAD, H)
    _, x2cat = _tc_pool(h2, st2, mask2, bcol, NPAD, H)

    out = _tc_final(x1cat, x2cat, lin1_W, l1br, l2r, l2br, H)
    return out[:, 0]
